# baseline (speedup 1.0000x reference)
import functools

import jax
import jax.numpy as jnp
from jax import lax
from jax.experimental import pallas as pl
from jax.experimental.pallas import tpu as pltpu
from jax.experimental.pallas import tpu_sc as plsc

BATCH = 8
TOTAL = 22500
PRE = 6000
POST = 300
THR = 0.7

NPAD = 22512
NB = NPAD // 16
DIG = 1024
MAXK = 0x7FFFFFFF
KBIAS = 0x3F800000


def _gat(ref, idx):
    return plsc.load_gather(ref, [idx])


def _sca(ref, idx, x):
    plsc.store_scatter(ref, [idx], x)


CH = 1200
NCH = PRE // CH


def _sc_sort_body(labels_hbm, deltas_hbm, anchors_hbm, out_hbm,
                  ka, kb, va, vb, cnt, sem, *rest):
    idxf = rest[:8]
    stg = rest[8:]
    wid = lax.axis_index("s") * 2 + lax.axis_index("c")

    @pl.when(wid < BATCH)
    def _():
        lanes = lax.iota(jnp.int32, 16)
        pltpu.sync_copy(labels_hbm.at[wid], ka)

        def zero_counts():
            def zbody(j, _):
                cnt[pl.ds(j * 16, 16)] = jnp.zeros((16,), jnp.int32)
                return 0
            lax.fori_loop(0, DIG * 16 // 16, zbody, 0)

        def key_from_f32(kf):
            bits = plsc.bitcast(kf, jnp.int32)
            k = KBIAS - bits
            return jnp.where(bits < 0, MAXK, k)

        def offsets_pass():
            def obody(j, run):
                v = cnt[pl.ds(j * 16, 16)]
                cs = plsc.cumsum(v)
                cnt[pl.ds(j * 16, 16)] = cs - v + run
                return run + jnp.sum(v)
            lax.fori_loop(0, DIG * 16 // 16, obody, jnp.int32(0))

        zero_counts()

        def c1(t, _):
            idx16 = lanes * NB + t
            k = key_from_f32(_gat(ka, idx16))
            ci = ((k & (DIG - 1)) * 16) + lanes
            _sca(cnt, ci, _gat(cnt, ci) + 1)
            return 0
        lax.fori_loop(0, NB, c1, 0)
        offsets_pass()

        def s1(t, _):
            idx16 = lanes * NB + t
            k = key_from_f32(_gat(ka, idx16))
            ci = ((k & (DIG - 1)) * 16) + lanes
            pos = _gat(cnt, ci)
            _sca(cnt, ci, pos + 1)
            _sca(kb, pos, k)
            _sca(vb, pos, idx16)
            return 0
        lax.fori_loop(0, NB, s1, 0)

        def load_i32(ref, idx):
            x = _gat(ref, idx)
            return plsc.bitcast(x, jnp.int32) if x.dtype != jnp.int32 else x

        def store_i32(ref, idx, x):
            if ref.dtype != jnp.int32:
                x = plsc.bitcast(x, ref.dtype)
            _sca(ref, idx, x)

        def radix_pass(sh, srck, srcv, dstk, dstv):
            zero_counts()

            def cN(t, _):
                idx16 = lanes * NB + t
                k = load_i32(srck, idx16)
                ci = (((k >> sh) & (DIG - 1)) * 16) + lanes
                _sca(cnt, ci, _gat(cnt, ci) + 1)
                return 0
            lax.fori_loop(0, NB, cN, 0)
            offsets_pass()

            def sN(t, _):
                idx16 = lanes * NB + t
                k = load_i32(srck, idx16)
                v = _gat(srcv, idx16)
                ci = (((k >> sh) & (DIG - 1)) * 16) + lanes
                pos = _gat(cnt, ci)
                _sca(cnt, ci, pos + 1)
                store_i32(dstk, pos, k)
                _sca(dstv, pos, v)
                return 0
            lax.fori_loop(0, NB, sN, 0)

        radix_pass(10, kb, vb, ka, va)
        radix_pass(20, ka, va, kb, vb)

        for j in range(9):
            padv = jnp.full((16,), DEAD if j == 0 else 0.0, jnp.float32)
            for t in range((PREPAD - PRE) // 16):
                stg[j][pl.ds(t * 16, 16)] = padv
        for j in range(9):
            pltpu.sync_copy(stg[j].at[pl.ds(0, PREPAD - PRE)],
                            out_hbm.at[pl.ds(wid * 9 * PREPAD + j * PREPAD + PRE, PREPAD - PRE)])

        for c in range(NCH):
            c0 = c * CH

            def ib(t, _):
                v = vb[pl.ds(c0 + t * 16, 16)]
                sl = pl.ds(t * 16, 16)
                db = (v + wid * TOTAL) * 4
                ab = v * 4
                for j in range(4):
                    idxf[j][sl] = db + j
                    idxf[4 + j][sl] = ab + j
                k = kb[pl.ds(c0 + t * 16, 16)]
                stg[0][sl] = plsc.bitcast(KBIAS - k, jnp.float32)
                return 0
            lax.fori_loop(0, CH // 16, ib, 0)
            copies = [
                pltpu.async_copy(deltas_hbm.at[idxf[j]], stg[1 + j], sem)
                for j in range(4)
            ] + [
                pltpu.async_copy(anchors_hbm.at[idxf[4 + j]], stg[5 + j], sem)
                for j in range(4)
            ]
            for d in copies:
                d.wait()
            for j in range(9):
                pltpu.sync_copy(stg[j], out_hbm.at[pl.ds(wid * 9 * PREPAD + j * PREPAD + c0, CH)])


def _sc_sort(labels_pad, deltas2, anchors):
    anchors = anchors.reshape(TOTAL * 4)
    mesh = plsc.VectorSubcoreMesh(core_axis_name="c", subcore_axis_name="s")
    f = functools.partial(
        pl.kernel,
        out_type=jax.ShapeDtypeStruct((BATCH * 9 * PREPAD,), jnp.float32),
        mesh=mesh,
        compiler_params=pltpu.CompilerParams(needs_layout_passes=False),
        scratch_types=[
            pltpu.VMEM((NPAD,), jnp.float32),
            pltpu.VMEM((NPAD,), jnp.int32),
            pltpu.VMEM((NPAD,), jnp.int32),
            pltpu.VMEM((NPAD,), jnp.int32),
            pltpu.VMEM((DIG * 16,), jnp.int32),
            pltpu.SemaphoreType.DMA,
        ] + [pltpu.VMEM((CH,), jnp.int32)] * 8
          + [pltpu.VMEM((CH,), jnp.float32)] * 9,
    )(_sc_sort_body)
    return f(labels_pad, deltas2, anchors)


W = 256
PREPAD = 6144
NEG = -1e9
DEAD = -1e10


def _nms_body(all_ref, oy1, ox1, oy2, ox2, y1_ref, x1_ref, y2_ref, x2_ref):
    def row(j):
        return all_ref[:, pl.ds(j * PREPAD, PREPAD)]

    ay1 = row(5)
    ax1 = row(6)
    ay2 = row(7)
    ax2 = row(8)
    anc_h = ay2 - ay1
    anc_w = ax2 - ax1
    anc_cy = ay1 + 0.5 * anc_h
    anc_cx = ax1 + 0.5 * anc_w
    bb_h = jnp.exp(row(3)) * anc_h
    bb_w = jnp.exp(row(4)) * anc_w
    bb_cy = row(1) * anc_h + anc_cy
    bb_cx = row(2) * anc_w + anc_cx
    y1_ref[...] = bb_cy - 0.5 * bb_h
    x1_ref[...] = bb_cx - 0.5 * bb_w
    y2_ref[...] = bb_cy + 0.5 * bb_h
    x2_ref[...] = bb_cx + 0.5 * bb_w

    wiota = lax.broadcasted_iota(jnp.int32, (BATCH, W), 1)
    oiota = lax.broadcasted_iota(jnp.int32, (BATCH, POST), 1)
    z = jnp.zeros((BATCH, POST), jnp.float32)
    kept0 = jnp.zeros((BATCH, 1), jnp.int32)

    def process_window(w, carry, first=False):
        kept, ay1, ax1, ay2, ax2, ky1, kx1, ky2, kx2, kar = carry
        start = pl.multiple_of(w * W, W)
        y1w = y1_ref[:, pl.ds(start, W)]
        x1w = x1_ref[:, pl.ds(start, W)]
        y2w = y2_ref[:, pl.ds(start, W)]
        x2w = x2_ref[:, pl.ds(start, W)]
        sw0 = all_ref[:, pl.ds(start, W)]
        areas = (y2w - y1w) * (x2w - x1w)

        def entry(sw):
            def ebody(k, sw):
                selk = oiota == k
                bky1 = jnp.sum(jnp.where(selk, ky1, 0.0), axis=1, keepdims=True)
                bkx1 = jnp.sum(jnp.where(selk, kx1, 0.0), axis=1, keepdims=True)
                bky2 = jnp.sum(jnp.where(selk, ky2, 0.0), axis=1, keepdims=True)
                bkx2 = jnp.sum(jnp.where(selk, kx2, 0.0), axis=1, keepdims=True)
                bkar = jnp.sum(jnp.where(selk, kar, 0.0), axis=1, keepdims=True)
                kvld = k < kept
                yy1 = jnp.maximum(bky1, y1w)
                xx1 = jnp.maximum(bkx1, x1w)
                yy2 = jnp.minimum(bky2, y2w)
                xx2 = jnp.minimum(bkx2, x2w)
                inter = jnp.maximum(yy2 - yy1, 0.0) * jnp.maximum(xx2 - xx1, 0.0)
                iou = inter / (bkar + areas - inter + 1e-8)
                return jnp.where(kvld & (iou >= THR), DEAD, sw)
            return lax.fori_loop(0, POST, ebody, sw)

        sw0 = sw0 if first else lax.cond(
            jnp.any(kept > 0), entry, lambda s: s, sw0)
        go0 = jnp.any((jnp.max(sw0, axis=1, keepdims=True) > NEG) & (kept < POST))

        def cond(c):
            return c[0]

        def body(c):
            _, sw, kept, ay1, ax1, ay2, ax2, ky1, kx1, ky2, kx2, kar = c
            idxv = jnp.min(jnp.where(sw > NEG, wiota, W), axis=1, keepdims=True)
            act = (idxv < W) & (kept < POST)
            sel = (wiota == idxv) & act
            by1 = jnp.sum(jnp.where(sel, y1w, 0.0), axis=1, keepdims=True)
            bx1 = jnp.sum(jnp.where(sel, x1w, 0.0), axis=1, keepdims=True)
            by2 = jnp.sum(jnp.where(sel, y2w, 0.0), axis=1, keepdims=True)
            bx2 = jnp.sum(jnp.where(sel, x2w, 0.0), axis=1, keepdims=True)
            bar = (by2 - by1) * (bx2 - bx1)
            yy1 = jnp.maximum(by1, y1w)
            xx1 = jnp.maximum(bx1, x1w)
            yy2 = jnp.minimum(by2, y2w)
            xx2 = jnp.minimum(bx2, x2w)
            inter = jnp.maximum(yy2 - yy1, 0.0) * jnp.maximum(xx2 - xx1, 0.0)
            iou = inter / (bar + areas - inter + 1e-8)
            supp = act & (iou >= THR)
            sw = jnp.where(supp | sel, DEAD, sw)
            slot = (oiota == kept) & act
            ay1 = jnp.where(slot, jnp.clip(by1, 0.0, 1.0), ay1)
            ax1 = jnp.where(slot, jnp.clip(bx1, 0.0, 1.0), ax1)
            ay2 = jnp.where(slot, jnp.clip(by2, 0.0, 1.0), ay2)
            ax2 = jnp.where(slot, jnp.clip(bx2, 0.0, 1.0), ax2)
            ky1 = jnp.where(slot, by1, ky1)
            kx1 = jnp.where(slot, bx1, kx1)
            ky2 = jnp.where(slot, by2, ky2)
            kx2 = jnp.where(slot, bx2, kx2)
            kar = jnp.where(slot, bar, kar)
            kept = kept + act.astype(jnp.int32)
            go = jnp.any((jnp.max(sw, axis=1, keepdims=True) > NEG)
                         & (kept < POST))
            return (go, sw, kept, ay1, ax1, ay2, ax2, ky1, kx1, ky2, kx2, kar)

        if first:
            def fbody(_, c):
                return body(c)
            c = lax.fori_loop(
                0, POST, fbody,
                (go0, sw0, kept, ay1, ax1, ay2, ax2, ky1, kx1, ky2, kx2, kar))
        else:
            c = lax.while_loop(
                cond, body,
                (go0, sw0, kept, ay1, ax1, ay2, ax2, ky1, kx1, ky2, kx2, kar))
        return c[2:]

    def window_step(w, carry):
        return lax.cond(
            jnp.any(carry[0] < POST),
            lambda c: process_window(w, c),
            lambda c: c,
            carry)

    carry = (kept0, z, z, z, z, z, z, z, z, z)
    carry = process_window(0, carry, first=True)
    carry = lax.fori_loop(1, PREPAD // W, window_step, carry)
    _, ay1, ax1, ay2, ax2 = carry[:5]
    oy1[...] = ay1
    ox1[...] = ax1
    oy2[...] = ay2
    ox2[...] = ax2


def kernel(rpn_bbox_deltas, rpn_labels, anchors):
    deltas2 = rpn_bbox_deltas.reshape(BATCH * TOTAL * 4)
    labels = rpn_labels.reshape(BATCH, TOTAL)
    labels_pad = jnp.concatenate(
        [labels, jnp.full((BATCH, NPAD - TOTAL), -1.0, jnp.float32)], axis=1)

    allarr = _sc_sort(labels_pad, deltas2, anchors).reshape(BATCH, 9 * PREPAD)
    outs = pl.pallas_call(
        _nms_body,
        out_shape=[jax.ShapeDtypeStruct((BATCH, POST), jnp.float32)] * 4,
        scratch_shapes=[pltpu.VMEM((BATCH, PREPAD), jnp.float32)] * 4,
    )(allarr)
    return jnp.stack(outs, axis=-1)


# speedup vs baseline: 1.1215x; 1.1215x over previous
import functools

import jax
import jax.numpy as jnp
from jax import lax
from jax.experimental import pallas as pl
from jax.experimental.pallas import tpu as pltpu
from jax.experimental.pallas import tpu_sc as plsc

BATCH = 8
TOTAL = 22500
PRE = 6000
POST = 300
THR = 0.7

NPAD = 22512
NB = NPAD // 16
DIG = 1024
MAXK = 0x7FFFFFFF
KBIAS = 0x3F800000


def _gat(ref, idx):
    return plsc.load_gather(ref, [idx])


def _sca(ref, idx, x):
    plsc.store_scatter(ref, [idx], x)


CH = 1200
NCH = PRE // CH


def _sc_sort_body(labels_hbm, deltas_hbm, anchors_hbm, out_hbm,
                  ka, kb, va, vb, cnt, sem, *rest):
    idxf = rest[:8]
    stg = rest[8:]
    wid = lax.axis_index("s") * 2 + lax.axis_index("c")

    @pl.when(wid < BATCH)
    def _():
        lanes = lax.iota(jnp.int32, 16)
        pltpu.sync_copy(labels_hbm.at[wid], ka)

        def zero_counts():
            def zbody(j, _):
                cnt[pl.ds(j * 16, 16)] = jnp.zeros((16,), jnp.int32)
                return 0
            lax.fori_loop(0, DIG * 16 // 16, zbody, 0)

        def key_from_f32(kf):
            bits = plsc.bitcast(kf, jnp.int32)
            k = KBIAS - bits
            return jnp.where(bits < 0, MAXK, k)

        def offsets_pass():
            def obody(j, run):
                v = cnt[pl.ds(j * 16, 16)]
                cs = plsc.cumsum(v)
                cnt[pl.ds(j * 16, 16)] = cs - v + run
                return run + jnp.sum(v)
            lax.fori_loop(0, DIG * 16 // 16, obody, jnp.int32(0))

        zero_counts()

        def c1(t, _):
            idx16 = lanes * NB + t
            k = key_from_f32(_gat(ka, idx16))
            ci = ((k & (DIG - 1)) * 16) + lanes
            _sca(cnt, ci, _gat(cnt, ci) + 1)
            return 0
        lax.fori_loop(0, NB, c1, 0)
        offsets_pass()

        def s1(t, _):
            idx16 = lanes * NB + t
            k = key_from_f32(_gat(ka, idx16))
            ci = ((k & (DIG - 1)) * 16) + lanes
            pos = _gat(cnt, ci)
            _sca(cnt, ci, pos + 1)
            _sca(kb, pos, k)
            _sca(vb, pos, idx16)
            return 0
        lax.fori_loop(0, NB, s1, 0)

        def load_i32(ref, idx):
            x = _gat(ref, idx)
            return plsc.bitcast(x, jnp.int32) if x.dtype != jnp.int32 else x

        def store_i32(ref, idx, x):
            if ref.dtype != jnp.int32:
                x = plsc.bitcast(x, ref.dtype)
            _sca(ref, idx, x)

        def radix_pass(sh, srck, srcv, dstk, dstv):
            zero_counts()

            def cN(t, _):
                idx16 = lanes * NB + t
                k = load_i32(srck, idx16)
                ci = (((k >> sh) & (DIG - 1)) * 16) + lanes
                _sca(cnt, ci, _gat(cnt, ci) + 1)
                return 0
            lax.fori_loop(0, NB, cN, 0)
            offsets_pass()

            def sN(t, _):
                idx16 = lanes * NB + t
                k = load_i32(srck, idx16)
                v = _gat(srcv, idx16)
                ci = (((k >> sh) & (DIG - 1)) * 16) + lanes
                pos = _gat(cnt, ci)
                _sca(cnt, ci, pos + 1)
                store_i32(dstk, pos, k)
                _sca(dstv, pos, v)
                return 0
            lax.fori_loop(0, NB, sN, 0)

        radix_pass(10, kb, vb, ka, va)
        radix_pass(20, ka, va, kb, vb)

        for j in range(9):
            padv = jnp.full((16,), DEAD if j == 0 else 0.0, jnp.float32)
            for t in range((PREPAD - PRE) // 16):
                stg[j][pl.ds(t * 16, 16)] = padv
        for j in range(9):
            pltpu.sync_copy(stg[j].at[pl.ds(0, PREPAD - PRE)],
                            out_hbm.at[pl.ds(wid * 9 * PREPAD + j * PREPAD + PRE, PREPAD - PRE)])

        for c in range(NCH):
            c0 = c * CH

            def ib(t, _):
                v = vb[pl.ds(c0 + t * 16, 16)]
                sl = pl.ds(t * 16, 16)
                db = (v + wid * TOTAL) * 4
                ab = v * 4
                for j in range(4):
                    idxf[j][sl] = db + j
                    idxf[4 + j][sl] = ab + j
                k = kb[pl.ds(c0 + t * 16, 16)]
                stg[0][sl] = plsc.bitcast(KBIAS - k, jnp.float32)
                return 0
            lax.fori_loop(0, CH // 16, ib, 0)
            copies = [
                pltpu.async_copy(deltas_hbm.at[idxf[j]], stg[1 + j], sem)
                for j in range(4)
            ] + [
                pltpu.async_copy(anchors_hbm.at[idxf[4 + j]], stg[5 + j], sem)
                for j in range(4)
            ]
            for d in copies:
                d.wait()
            for j in range(9):
                pltpu.sync_copy(stg[j], out_hbm.at[pl.ds(wid * 9 * PREPAD + j * PREPAD + c0, CH)])


def _sc_sort(labels_pad, deltas2, anchors):
    anchors = anchors.reshape(TOTAL * 4)
    mesh = plsc.VectorSubcoreMesh(core_axis_name="c", subcore_axis_name="s")
    f = functools.partial(
        pl.kernel,
        out_type=jax.ShapeDtypeStruct((BATCH * 9 * PREPAD,), jnp.float32),
        mesh=mesh,
        compiler_params=pltpu.CompilerParams(needs_layout_passes=False),
        scratch_types=[
            pltpu.VMEM((NPAD,), jnp.float32),
            pltpu.VMEM((NPAD,), jnp.int32),
            pltpu.VMEM((NPAD,), jnp.int32),
            pltpu.VMEM((NPAD,), jnp.int32),
            pltpu.VMEM((DIG * 16,), jnp.int32),
            pltpu.SemaphoreType.DMA,
        ] + [pltpu.VMEM((CH,), jnp.int32)] * 8
          + [pltpu.VMEM((CH,), jnp.float32)] * 9,
    )(_sc_sort_body)
    return f(labels_pad, deltas2, anchors)


W = 512
PREPAD = 6144
NEG = -1e9
DEAD = -1e10


def _nms_body(all_ref, oy1, ox1, oy2, ox2, y1_ref, x1_ref, y2_ref, x2_ref):
    def row(j):
        return all_ref[:, pl.ds(j * PREPAD, PREPAD)]

    ay1 = row(5)
    ax1 = row(6)
    ay2 = row(7)
    ax2 = row(8)
    anc_h = ay2 - ay1
    anc_w = ax2 - ax1
    anc_cy = ay1 + 0.5 * anc_h
    anc_cx = ax1 + 0.5 * anc_w
    bb_h = jnp.exp(row(3)) * anc_h
    bb_w = jnp.exp(row(4)) * anc_w
    bb_cy = row(1) * anc_h + anc_cy
    bb_cx = row(2) * anc_w + anc_cx
    y1_ref[...] = bb_cy - 0.5 * bb_h
    x1_ref[...] = bb_cx - 0.5 * bb_w
    y2_ref[...] = bb_cy + 0.5 * bb_h
    x2_ref[...] = bb_cx + 0.5 * bb_w

    wiota = lax.broadcasted_iota(jnp.int32, (BATCH, W), 1)
    oiota = lax.broadcasted_iota(jnp.int32, (BATCH, POST), 1)
    z = jnp.zeros((BATCH, POST), jnp.float32)
    kept0 = jnp.zeros((BATCH, 1), jnp.int32)

    def process_window(w, carry, first=False):
        kept, ky1, kx1, ky2, kx2 = carry
        start = pl.multiple_of(w * W, W)
        y1w = y1_ref[:, pl.ds(start, W)]
        x1w = x1_ref[:, pl.ds(start, W)]
        y2w = y2_ref[:, pl.ds(start, W)]
        x2w = x2_ref[:, pl.ds(start, W)]
        sw0 = all_ref[:, pl.ds(start, W)]
        areas = (y2w - y1w) * (x2w - x1w)

        def entry(sw):
            def ebody(k, sw):
                selk = oiota == k
                bky1 = jnp.sum(jnp.where(selk, ky1, 0.0), axis=1, keepdims=True)
                bkx1 = jnp.sum(jnp.where(selk, kx1, 0.0), axis=1, keepdims=True)
                bky2 = jnp.sum(jnp.where(selk, ky2, 0.0), axis=1, keepdims=True)
                bkx2 = jnp.sum(jnp.where(selk, kx2, 0.0), axis=1, keepdims=True)
                bkar = (bky2 - bky1) * (bkx2 - bkx1)
                kvld = k < kept
                yy1 = jnp.maximum(bky1, y1w)
                xx1 = jnp.maximum(bkx1, x1w)
                yy2 = jnp.minimum(bky2, y2w)
                xx2 = jnp.minimum(bkx2, x2w)
                inter = jnp.maximum(yy2 - yy1, 0.0) * jnp.maximum(xx2 - xx1, 0.0)
                iou = inter / (bkar + areas - inter + 1e-8)
                return jnp.where(kvld & (iou >= THR), DEAD, sw)
            return lax.fori_loop(0, POST, ebody, sw)

        sw0 = sw0 if first else lax.cond(
            jnp.any(kept > 0), entry, lambda s: s, sw0)
        go0 = jnp.any((jnp.max(sw0, axis=1, keepdims=True) > NEG) & (kept < POST))

        def cond(c):
            return c[0]

        def body(c):
            _, sw, kept, ky1, kx1, ky2, kx2 = c
            idxv = jnp.min(jnp.where(sw > NEG, wiota, W), axis=1, keepdims=True)
            act = (idxv < W) & (kept < POST)
            sel = (wiota == idxv) & act
            by1 = jnp.sum(jnp.where(sel, y1w, 0.0), axis=1, keepdims=True)
            bx1 = jnp.sum(jnp.where(sel, x1w, 0.0), axis=1, keepdims=True)
            by2 = jnp.sum(jnp.where(sel, y2w, 0.0), axis=1, keepdims=True)
            bx2 = jnp.sum(jnp.where(sel, x2w, 0.0), axis=1, keepdims=True)
            bar = (by2 - by1) * (bx2 - bx1)
            yy1 = jnp.maximum(by1, y1w)
            xx1 = jnp.maximum(bx1, x1w)
            yy2 = jnp.minimum(by2, y2w)
            xx2 = jnp.minimum(bx2, x2w)
            inter = jnp.maximum(yy2 - yy1, 0.0) * jnp.maximum(xx2 - xx1, 0.0)
            iou = inter / (bar + areas - inter + 1e-8)
            supp = act & (iou >= THR)
            sw = jnp.where(supp | sel, DEAD, sw)
            slot = (oiota == kept) & act
            ky1 = jnp.where(slot, by1, ky1)
            kx1 = jnp.where(slot, bx1, kx1)
            ky2 = jnp.where(slot, by2, ky2)
            kx2 = jnp.where(slot, bx2, kx2)
            kept = kept + act.astype(jnp.int32)
            go = jnp.any((jnp.max(sw, axis=1, keepdims=True) > NEG)
                         & (kept < POST))
            return (go, sw, kept, ky1, kx1, ky2, kx2)

        if first:
            def fbody(_, c):
                return body(c)
            c = lax.fori_loop(
                0, POST, fbody,
                (go0, sw0, kept, ky1, kx1, ky2, kx2))
        else:
            c = lax.while_loop(
                cond, body,
                (go0, sw0, kept, ky1, kx1, ky2, kx2))
        return c[2:]

    def window_step(w, carry):
        return lax.cond(
            jnp.any(carry[0] < POST),
            lambda c: process_window(w, c),
            lambda c: c,
            carry)

    carry = (kept0, z, z, z, z)
    carry = process_window(0, carry, first=True)
    carry = lax.fori_loop(1, PREPAD // W, window_step, carry)
    _, ky1, kx1, ky2, kx2 = carry
    oy1[...] = jnp.clip(ky1, 0.0, 1.0)
    ox1[...] = jnp.clip(kx1, 0.0, 1.0)
    oy2[...] = jnp.clip(ky2, 0.0, 1.0)
    ox2[...] = jnp.clip(kx2, 0.0, 1.0)


def kernel(rpn_bbox_deltas, rpn_labels, anchors):
    deltas2 = rpn_bbox_deltas.reshape(BATCH * TOTAL * 4)
    labels = rpn_labels.reshape(BATCH, TOTAL)
    labels_pad = jnp.concatenate(
        [labels, jnp.full((BATCH, NPAD - TOTAL), -1.0, jnp.float32)], axis=1)

    allarr = _sc_sort(labels_pad, deltas2, anchors).reshape(BATCH, 9 * PREPAD)
    outs = pl.pallas_call(
        _nms_body,
        out_shape=[jax.ShapeDtypeStruct((BATCH, POST), jnp.float32)] * 4,
        scratch_shapes=[pltpu.VMEM((BATCH, PREPAD), jnp.float32)] * 4,
    )(allarr)
    return jnp.stack(outs, axis=-1)
